# baseline (speedup 1.0000x reference)
import functools

import jax
import jax.numpy as jnp
from jax import lax
from jax.experimental import pallas as pl
from jax.experimental.pallas import tpu as pltpu
from jax.experimental.pallas import tpu_sc as plsc

_N_EXPERTS = 64
_DIM = 768
_TOKENS = 32768
_NW = 32
_CHUNK = _TOKENS // _NW
_L = 16


def _mm_block(x_ref, w_ref, o_ref):
    st = lax.dot_general(
        w_ref[...], x_ref[...], (((1,), (1,)), ((), ())),
        preferred_element_type=jnp.float32,
    )
    o_ref[0] = jax.nn.sigmoid(st)


def _scores_chunked(x, weight):
    return pl.pallas_call(
        _mm_block,
        grid=(_NW,),
        in_specs=[
            pl.BlockSpec((_CHUNK, _DIM), lambda i: (i, 0)),
            pl.BlockSpec((_N_EXPERTS, _DIM), lambda i: (0, 0)),
        ],
        out_specs=pl.BlockSpec((1, _N_EXPERTS, _CHUNK), lambda i: (i, 0, 0)),
        out_shape=jax.ShapeDtypeStruct((_NW, _N_EXPERTS, _CHUNK), jnp.float32),
        compiler_params=pltpu.CompilerParams(dimension_semantics=("arbitrary",)),
    )(x, weight)


@functools.partial(
    pl.kernel,
    mesh=plsc.VectorSubcoreMesh(core_axis_name="c", subcore_axis_name="s"),
    out_type=[
        jax.ShapeDtypeStruct((_NW, 8, _CHUNK), jnp.float32),
        jax.ShapeDtypeStruct((_NW, 8, _CHUNK), jnp.int32),
    ],
    scratch_types=[
        pltpu.VMEM((_N_EXPERTS, _CHUNK), jnp.float32),
        pltpu.VMEM((8, _CHUNK), jnp.float32),
        pltpu.VMEM((8, _CHUNK), jnp.int32),
    ],
)
def _route_sc(scores_hbm, wout_hbm, iout_hbm, s_v, w_v, i_v):
    wid = lax.axis_index("s") * 2 + lax.axis_index("c")
    pltpu.sync_copy(scores_hbm.at[wid], s_v)

    def body(j, carry):
        off = j * _L
        s = [s_v[e, pl.ds(off, _L)] for e in range(_N_EXPERTS)]

        gs = []
        for g in range(8):
            b = 8 * g
            m1, m2 = [], []
            for p in range(4):
                a, c = s[b + 2 * p], s[b + 2 * p + 1]
                m1.append(jnp.maximum(a, c))
                m2.append(jnp.minimum(a, c))
            while len(m1) > 1:
                n1, n2 = [], []
                for p in range(0, len(m1), 2):
                    n1.append(jnp.maximum(m1[p], m1[p + 1]))
                    n2.append(
                        jnp.maximum(
                            jnp.minimum(m1[p], m1[p + 1]),
                            jnp.maximum(m2[p], m2[p + 1]),
                        )
                    )
                m1, m2 = n1, n2
            gs.append(m1[0] + m2[0])

        ge = {}
        for a in range(8):
            for c in range(a + 1, 8):
                one = jnp.full((_L,), 1, jnp.int32)
                zero = jnp.full((_L,), 0, jnp.int32)
                ge[(a, c)] = jnp.where(gs[a] >= gs[c], one, zero)
        sel = []
        for g in range(8):
            r = jnp.zeros((_L,), jnp.int32)
            for a in range(g):
                r = r + ge[(a, g)]
            for c in range(g + 1, 8):
                r = r + 1 - ge[(g, c)]
            sel.append(r < 4)
        negv = jnp.full((_L,), float("-inf"), jnp.float32)
        m = [jnp.where(sel[e // 8], s[e], negv) for e in range(_N_EXPERTS)]

        vals, idxs = [], []
        for k in range(8):
            tv = list(m)
            ti = [jnp.full((_L,), e, jnp.int32) for e in range(_N_EXPERTS)]
            while len(tv) > 1:
                nv, ni = [], []
                for p in range(0, len(tv), 2):
                    cond = tv[p] >= tv[p + 1]
                    nv.append(jnp.where(cond, tv[p], tv[p + 1]))
                    ni.append(jnp.where(cond, ti[p], ti[p + 1]))
                tv, ti = nv, ni
            vals.append(tv[0])
            idxs.append(ti[0])
            if k < 7:
                m = [
                    jnp.where(ti[0] == e, negv, m[e])
                    for e in range(_N_EXPERTS)
                ]

        tot = vals[0]
        for k in range(1, 8):
            tot = tot + vals[k]
        scale = 2.5 / tot
        for k in range(8):
            w_v[k, pl.ds(off, _L)] = vals[k] * scale
            i_v[k, pl.ds(off, _L)] = idxs[k]
        return carry

    lax.fori_loop(0, _CHUNK // _L, body, 0)
    pltpu.sync_copy(w_v, wout_hbm.at[wid])
    pltpu.sync_copy(i_v, iout_hbm.at[wid])


@jax.jit
def kernel(x, weight):
    scores = _scores_chunked(x, weight)
    wc, ic = _route_sc(scores)
    wts = wc.transpose(0, 2, 1).reshape(_TOKENS, 8)
    idx = ic.transpose(0, 2, 1).reshape(_TOKENS, 8)
    return wts, idx
